# Initial kernel scaffold; baseline (speedup 1.0000x reference)
import jax
import jax.numpy as jnp
from jax import lax
from jax.experimental import pallas as pl
from jax.experimental.pallas import tpu as pltpu
from jax.experimental.pallas import tpu_sc as plsc

N = 10000
E = 320000
FD = 256
D = 128
NR = 200

NC = 2
NS = 16

EPW = 20480
NCH_A = EPW // 128
ROWS_A = 10112
SLAB_A = ROWS_A // NS
TRASH_A = 10016

EC = 655360
CCH = 2048
NCROW = EC // CCH
CPS = NCROW // NS
ACC_C = 1280128
TRASH_C = 1280000
CSLAB = ACC_C // NS
COUT = N * 128


def _sc_count_kernel(dstc, colc, zeros_c, out, acc, dbuf, cbuf, ibuf, ones):
    c = lax.axis_index("c")
    s = lax.axis_index("s")

    for g in range(8):
        ones[pl.ds(g * 16, 16), 0] = jnp.ones((16,), jnp.float32)

    for p in range(2):
        r128 = (c * 2 + p) * 128
        zb = s * CSLAB
        for z in range(9):
            pltpu.sync_copy(zeros_c.at[pl.ds(z * 8192, 8192)],
                            acc.at[pl.ds(zb + z * 8192, 8192)])
        pltpu.sync_copy(zeros_c.at[pl.ds(0, CSLAB - 9 * 8192)],
                        acc.at[pl.ds(zb + 9 * 8192, CSLAB - 9 * 8192)])
        plsc.subcore_barrier()

        def chunk_body(k, carry):
            row = s * CPS + k
            pltpu.sync_copy(dstc.at[row], dbuf)
            pltpu.sync_copy(colc.at[row], cbuf)
            for g in range(128):
                d = dbuf[pl.ds(g * 16, 16)]
                cl = cbuf[pl.ds(g * 16, 16)]
                t = cl - r128
                ok = (t >= 0) & (t < 128)
                idx = jnp.where(ok, d * 128 + t, TRASH_C)
                ibuf[g // 8, pl.ds((g % 8) * 16, 16)] = idx
            for rr in range(16):
                pltpu.sync_copy(ones, acc.at[ibuf.at[rr]], add=True)
            return carry

        lax.fori_loop(0, CPS, chunk_body, 0)
        plsc.subcore_barrier()
        ob = (c * 2 + p) * COUT + s * (COUT // NS)
        ab = s * (COUT // NS)
        for z in range(9):
            pltpu.sync_copy(acc.at[pl.ds(ab + z * 8192, 8192)],
                            out.at[pl.ds(ob + z * 8192, 8192)])
        rem = COUT // NS - 9 * 8192
        pltpu.sync_copy(acc.at[pl.ds(ab + 9 * 8192, rem)],
                        out.at[pl.ds(ob + 9 * 8192, rem)])
        plsc.subcore_barrier()


def _sc_accum_kernel(h, srcs, dsts, zeros_a, out, acc, sidx, didx, gbuf, sem):
    c = lax.axis_index("c")
    s = lax.axis_index("s")

    pltpu.sync_copy(zeros_a.at[pl.ds(s * SLAB_A, SLAB_A)],
                    acc.at[pl.ds(s * SLAB_A, SLAB_A)])
    rbase = (c * NS + s) * NCH_A
    pltpu.sync_copy(srcs.at[pl.ds(rbase, NCH_A)], sidx)
    pltpu.sync_copy(dsts.at[pl.ds(rbase, NCH_A)], didx)
    plsc.subcore_barrier()

    def body(j, carry):
        pltpu.async_copy(h.at[sidx.at[j]], gbuf, sem).wait()
        pltpu.sync_copy(gbuf, acc.at[didx.at[j]], add=True)
        return carry

    lax.fori_loop(0, NCH_A, body, 0)
    plsc.subcore_barrier()

    ob = c * ROWS_A + s * SLAB_A
    pltpu.sync_copy(acc.at[pl.ds(s * SLAB_A, SLAB_A)],
                    out.at[pl.ds(ob, SLAB_A)])


def _sc_count(dstc, colc, zeros_c):
    return pl.kernel(
        _sc_count_kernel,
        out_type=jax.ShapeDtypeStruct((4 * COUT, 1), jnp.float32),
        mesh=plsc.VectorSubcoreMesh(core_axis_name="c", subcore_axis_name="s"),
        scratch_types=[
            pltpu.VMEM_SHARED((ACC_C, 1), jnp.float32),
            pltpu.VMEM((CCH,), jnp.int32),
            pltpu.VMEM((CCH,), jnp.int32),
            pltpu.VMEM((16, 128), jnp.int32),
            pltpu.VMEM((128, 1), jnp.float32),
        ],
    )(dstc, colc, zeros_c)


def _sc_accum(h, srcs, dsts, zeros_a):
    return pl.kernel(
        _sc_accum_kernel,
        out_type=jax.ShapeDtypeStruct((2 * ROWS_A, D), jnp.float32),
        mesh=plsc.VectorSubcoreMesh(core_axis_name="c", subcore_axis_name="s"),
        scratch_types=[
            pltpu.VMEM_SHARED((ROWS_A, D), jnp.float32),
            pltpu.VMEM((NCH_A, 128), jnp.int32),
            pltpu.VMEM((NCH_A, 128), jnp.int32),
            pltpu.VMEM((128, D), jnp.float32),
            pltpu.SemaphoreType.DMA,
        ],
    )(h, srcs, dsts, zeros_a)


def _feat_kernel(x_ref, w_ref, b_ref, ra0_ref, wr0_ref, wr1_ref, l1_ref,
                 h_ref, ra1_ref, rout_ref):
    h_ref[...] = jnp.dot(x_ref[...], w_ref[...],
                         preferred_element_type=jnp.float32) + b_ref[0:1, :]
    ra1 = jnp.dot(ra0_ref[...], wr0_ref[...], preferred_element_type=jnp.float32)
    rows = lax.broadcasted_iota(jnp.int32, ra1.shape, 0)
    ra1 = jnp.where(rows == 401, l1_ref[0:1, :], ra1)
    ra1_ref[...] = ra1
    rout_ref[...] = jnp.dot(ra1, wr1_ref[...], preferred_element_type=jnp.float32)


def _combine_kernel(ain_ref, aout_ref, cin_ref, cout_ref, h_ref,
                    relin_ref, relout_ref, win_ref, wout_ref, wloop_ref,
                    sm_ref, pre_ref, stats_ref, acc):
    i = pl.program_id(0)

    @pl.when(i == 0)
    def _():
        acc[...] = jnp.zeros_like(acc)

    cin = cin_ref[...]
    cout = cout_ref[...]
    rin = jnp.dot(cin, relin_ref[...], preferred_element_type=jnp.float32)
    rout = jnp.dot(cout, relout_ref[...], preferred_element_type=jnp.float32)
    ni = jnp.maximum(jnp.sum(cin, axis=1, keepdims=True), 1.0)
    no = jnp.maximum(jnp.sum(cout, axis=1, keepdims=True), 1.0)
    in_res = jnp.dot((ain_ref[...] - rin) / ni, win_ref[...],
                     preferred_element_type=jnp.float32)
    out_res = jnp.dot((aout_ref[...] - rout) / no, wout_ref[...],
                      preferred_element_type=jnp.float32)
    loop_res = jnp.dot(h_ref[...] - sm_ref[0:1, :], wloop_ref[...],
                       preferred_element_type=jnp.float32)
    pre = (in_res + out_res + loop_res) / 3.0 + sm_ref[1:2, :]
    pre_ref[...] = pre
    acc[0:1, :] += jnp.sum(pre, axis=0, keepdims=True)
    acc[1:2, :] += jnp.sum(pre * pre, axis=0, keepdims=True)

    @pl.when(i == pl.num_programs(0) - 1)
    def _():
        stats_ref[...] = acc[...]


def _bn_kernel(pre_ref, stats_ref, sm_ref, h_ref):
    mu = stats_ref[0:1, :] / float(N)
    ex2 = stats_ref[1:2, :] / float(N)
    var = ex2 - mu * mu
    inv = lax.rsqrt(var + 1e-5)
    h = (pre_ref[...] - mu) * inv * sm_ref[2:3, :] + sm_ref[3:4, :]
    h_ref[...] = jnp.maximum(h, 0.0)


def _tc_feat(x, feat_W, featb8, ra0p, wr0, wr1, l18):
    return pl.pallas_call(
        _feat_kernel,
        out_shape=(
            jax.ShapeDtypeStruct((N, D), jnp.float32),
            jax.ShapeDtypeStruct((408, D), jnp.float32),
            jax.ShapeDtypeStruct((408, D), jnp.float32),
        ),
    )(x, feat_W, featb8, ra0p, wr0, wr1, l18)


def _tc_combine(ain, aout, cin, cout, h, relin, relout, win, wout, wloop, sm):
    nb = 10
    blk = N // nb
    row = lambda i: (i, 0)
    full = lambda i: (0, 0)
    return pl.pallas_call(
        _combine_kernel,
        grid=(nb,),
        in_specs=[
            pl.BlockSpec((blk, D), row),
            pl.BlockSpec((blk, D), row),
            pl.BlockSpec((blk, 256), row),
            pl.BlockSpec((blk, 256), row),
            pl.BlockSpec((blk, D), row),
            pl.BlockSpec((256, D), full),
            pl.BlockSpec((256, D), full),
            pl.BlockSpec((D, D), full),
            pl.BlockSpec((D, D), full),
            pl.BlockSpec((D, D), full),
            pl.BlockSpec((8, D), full),
        ],
        out_specs=[
            pl.BlockSpec((blk, D), row),
            pl.BlockSpec((8, D), full),
        ],
        out_shape=(
            jax.ShapeDtypeStruct((N, D), jnp.float32),
            jax.ShapeDtypeStruct((8, D), jnp.float32),
        ),
        scratch_shapes=[pltpu.VMEM((8, D), jnp.float32)],
    )(ain, aout, cin, cout, h, relin, relout, win, wout, wloop, sm)


def _tc_bn(pre, stats, sm):
    return pl.pallas_call(
        _bn_kernel,
        out_shape=jax.ShapeDtypeStruct((N, D), jnp.float32),
    )(pre, stats, sm)


def _pack8(*rows):
    out = [jnp.reshape(r, (1, D)) for r in rows]
    out += [jnp.zeros((1, D), jnp.float32)] * (8 - len(out))
    return jnp.concatenate(out, axis=0)


def kernel(x, edge_index, edge_type, quals, feat_W, feat_b, init_rel,
           conv0_w_in, conv0_w_out, conv0_w_loop, conv0_w_rel, conv0_loop_rel,
           conv0_bias, conv0_gamma, conv0_beta,
           conv1_w_in, conv1_w_out, conv1_w_loop, conv1_w_rel, conv1_loop_rel,
           conv1_bias, conv1_gamma, conv1_beta):
    del quals
    ei0 = edge_index[0].astype(jnp.int32)
    ei1 = edge_index[1].astype(jnp.int32)
    et = edge_type.astype(jnp.int32)

    pad = EPW * NS - E
    z_pad = jnp.zeros((pad,), jnp.int32)
    t_pad = jnp.full((pad,), TRASH_A, jnp.int32)
    srcs = jnp.concatenate([ei0, z_pad, ei1, z_pad]).reshape(2 * NS * NCH_A, 128)
    dsts = jnp.concatenate([ei1, t_pad, ei0, t_pad]).reshape(2 * NS * NCH_A, 128)

    padc = EC - 2 * E
    dstc = jnp.concatenate([ei1, ei0, jnp.zeros((padc,), jnp.int32)])
    colc = jnp.concatenate([et, et + 256, jnp.full((padc,), 4096, jnp.int32)])
    dstc = dstc.reshape(NCROW, CCH)
    colc = colc.reshape(NCROW, CCH)

    zeros_a = jnp.zeros((ROWS_A, D), jnp.float32)
    zeros_c = jnp.zeros((8192, 1), jnp.float32)

    ra0p = jnp.concatenate(
        [init_rel, conv0_loop_rel, jnp.zeros((6, D), jnp.float32)], axis=0)
    featb8 = _pack8(feat_b)
    l18 = _pack8(conv1_loop_rel[0])

    h0, ra1p, routp = _tc_feat(x, feat_W, featb8, ra0p, conv0_w_rel,
                               conv1_w_rel, l18)

    cflat = _sc_count(dstc, colc, zeros_c)
    c4 = cflat.reshape(4, N, 128)
    cin = jnp.concatenate([c4[0], c4[1]], axis=1)
    cout = jnp.concatenate([c4[2], c4[3]], axis=1)

    def rel_slices(rap):
        relin = jnp.concatenate(
            [rap[:200], jnp.zeros((56, D), jnp.float32)], axis=0)
        relout = jnp.concatenate(
            [rap[200:400], jnp.zeros((56, D), jnp.float32)], axis=0)
        return relin, relout

    h = h0
    rap = ra0p
    params = [
        (conv0_w_in, conv0_w_out, conv0_w_loop, conv0_loop_rel, conv0_bias,
         conv0_gamma, conv0_beta),
        (conv1_w_in, conv1_w_out, conv1_w_loop, conv1_w_rel, conv1_loop_rel,
         conv1_bias, conv1_gamma, conv1_beta),
    ]
    for prm in params:
        w_in, w_out, w_loop, loop_rel, bias, gamma, beta = prm[0], prm[1], prm[2], prm[-4], prm[-3], prm[-2], prm[-1]
        a = _sc_accum(h, srcs, dsts, zeros_a)
        ain = a[:N]
        aout = a[ROWS_A:ROWS_A + N]
        relin, relout = rel_slices(rap)
        sm = _pack8(loop_rel[0], bias, gamma, beta)
        pre, stats = _tc_combine(ain, aout, cin, cout, h, relin, relout,
                                 w_in, w_out, w_loop, sm)
        h = _tc_bn(pre, stats, sm)
        rap = ra1p

    return (h, routp[:401])


# speedup vs baseline: 2.1418x; 2.1418x over previous
import jax
import jax.numpy as jnp
from jax import lax
from jax.experimental import pallas as pl
from jax.experimental.pallas import tpu as pltpu
from jax.experimental.pallas import tpu_sc as plsc

N = 10000
E = 320000
FD = 256
D = 128
NR = 200

NC = 2
NS = 16

EPW = 20480
NCH = EPW // 128
ROWS_A = 10112
SLAB = ROWS_A // NS
TRASH = 10016
GREL_PAD = 500


def _sc_accum_kernel(h, srcs, dsts, zeros_a, out, acc, sidx, didx, gbuf, sem):
    c = lax.axis_index("c")
    s = lax.axis_index("s")

    pltpu.sync_copy(zeros_a.at[pl.ds(s * SLAB, SLAB)],
                    acc.at[pl.ds(s * SLAB, SLAB)])
    rbase = (c * NS + s) * NCH
    plsc.subcore_barrier()

    def body(j, carry):
        pltpu.async_copy(h.at[sidx.at[j]], gbuf, sem).wait()
        pltpu.sync_copy(gbuf, acc.at[didx.at[j]], add=True)
        return carry

    half = NCH // 2
    for hh in range(2):
        pltpu.sync_copy(srcs.at[pl.ds(rbase + hh * half, half)], sidx)
        pltpu.sync_copy(dsts.at[pl.ds(rbase + hh * half, half)], didx)
        lax.fori_loop(0, half, body, 0)
    plsc.subcore_barrier()

    ob = c * ROWS_A + s * SLAB
    pltpu.sync_copy(acc.at[pl.ds(s * SLAB, SLAB)], out.at[pl.ds(ob, SLAB)])


def _sc_accum(h, srcs, dsts, zeros_a):
    return pl.kernel(
        _sc_accum_kernel,
        out_type=jax.ShapeDtypeStruct((2 * ROWS_A, D), jnp.float32),
        mesh=plsc.VectorSubcoreMesh(core_axis_name="c", subcore_axis_name="s"),
        scratch_types=[
            pltpu.VMEM_SHARED((ROWS_A, D), jnp.float32),
            pltpu.VMEM((NCH // 2, 128), jnp.int32),
            pltpu.VMEM((NCH // 2, 128), jnp.int32),
            pltpu.VMEM((128, D), jnp.float32),
            pltpu.SemaphoreType.DMA,
        ],
    )(h, srcs, dsts, zeros_a)


def _feat_kernel(x_ref, w_ref, b_ref, ra0_ref, wr0_ref, wr1_ref, l1_ref,
                 h_ref, ra1_ref, rout_ref):
    h_ref[...] = jnp.dot(x_ref[...], w_ref[...],
                         preferred_element_type=jnp.float32) + b_ref[0:1, :]
    ra1 = jnp.dot(ra0_ref[...], wr0_ref[...], preferred_element_type=jnp.float32)
    rows = lax.broadcasted_iota(jnp.int32, ra1.shape, 0)
    ra1 = jnp.where(rows == 401, l1_ref[0:1, :], ra1)
    ra1_ref[...] = ra1
    rout_ref[...] = jnp.dot(ra1, wr1_ref[...], preferred_element_type=jnp.float32)


def _combine_kernel(ain_ref, aout_ref, rin_ref, rout_ref, cin_ref, cout_ref,
                    h_ref, win_ref, wout_ref, wloop_ref,
                    sm_ref, pre_ref, stats_ref, acc):
    i = pl.program_id(0)

    @pl.when(i == 0)
    def _():
        acc[...] = jnp.zeros_like(acc)

    ni = jnp.maximum(cin_ref[:, 0:1], 1.0)
    no = jnp.maximum(cout_ref[:, 0:1], 1.0)
    in_res = jnp.dot((ain_ref[...] - rin_ref[...]) / ni, win_ref[...],
                     preferred_element_type=jnp.float32)
    out_res = jnp.dot((aout_ref[...] - rout_ref[...]) / no, wout_ref[...],
                      preferred_element_type=jnp.float32)
    loop_res = jnp.dot(h_ref[...] - sm_ref[0:1, :], wloop_ref[...],
                       preferred_element_type=jnp.float32)
    pre = (in_res + out_res + loop_res) / 3.0 + sm_ref[1:2, :]
    pre_ref[...] = pre
    acc[0:1, :] += jnp.sum(pre, axis=0, keepdims=True)
    acc[1:2, :] += jnp.sum(pre * pre, axis=0, keepdims=True)

    @pl.when(i == pl.num_programs(0) - 1)
    def _():
        stats_ref[...] = acc[...]


def _relstep_kernel(r_ref, w_ref, o_ref):
    o_ref[...] = jnp.dot(r_ref[...], w_ref[...],
                         preferred_element_type=jnp.float32)


def _bn_kernel(pre_ref, stats_ref, sm_ref, h_ref):
    mu = stats_ref[0:1, :] / float(N)
    ex2 = stats_ref[1:2, :] / float(N)
    var = ex2 - mu * mu
    inv = lax.rsqrt(var + 1e-5)
    h = (pre_ref[...] - mu) * inv * sm_ref[2:3, :] + sm_ref[3:4, :]
    h_ref[...] = jnp.maximum(h, 0.0)


def _tc_feat(x, feat_W, featb8, ra0p, wr0, wr1, l18):
    return pl.pallas_call(
        _feat_kernel,
        out_shape=(
            jax.ShapeDtypeStruct((N, D), jnp.float32),
            jax.ShapeDtypeStruct((408, D), jnp.float32),
            jax.ShapeDtypeStruct((408, D), jnp.float32),
        ),
    )(x, feat_W, featb8, ra0p, wr0, wr1, l18)


def _tc_combine(ain, aout, rin, rout, cin, cout, h, win, wout, wloop, sm):
    nb = 10
    blk = N // nb
    row = lambda i: (i, 0)
    full = lambda i: (0, 0)
    return pl.pallas_call(
        _combine_kernel,
        grid=(nb,),
        in_specs=[
            pl.BlockSpec((blk, D), row),
            pl.BlockSpec((blk, D), row),
            pl.BlockSpec((blk, D), row),
            pl.BlockSpec((blk, D), row),
            pl.BlockSpec((blk, D), row),
            pl.BlockSpec((blk, D), row),
            pl.BlockSpec((blk, D), row),
            pl.BlockSpec((D, D), full),
            pl.BlockSpec((D, D), full),
            pl.BlockSpec((D, D), full),
            pl.BlockSpec((8, D), full),
        ],
        out_specs=[
            pl.BlockSpec((blk, D), row),
            pl.BlockSpec((8, D), full),
        ],
        out_shape=(
            jax.ShapeDtypeStruct((N, D), jnp.float32),
            jax.ShapeDtypeStruct((8, D), jnp.float32),
        ),
        scratch_shapes=[pltpu.VMEM((8, D), jnp.float32)],
    )(ain, aout, rin, rout, cin, cout, h, win, wout, wloop, sm)


def _tc_relstep(r2, w):
    return pl.pallas_call(
        _relstep_kernel,
        out_shape=jax.ShapeDtypeStruct((2 * ROWS_A, D), jnp.float32),
    )(r2, w)


def _tc_bn(pre, stats, sm):
    return pl.pallas_call(
        _bn_kernel,
        out_shape=jax.ShapeDtypeStruct((N, D), jnp.float32),
    )(pre, stats, sm)


def _pack8(*rows):
    out = [jnp.reshape(r, (1, D)) for r in rows]
    out += [jnp.zeros((1, D), jnp.float32)] * (8 - len(out))
    return jnp.concatenate(out, axis=0)


def kernel(x, edge_index, edge_type, quals, feat_W, feat_b, init_rel,
           conv0_w_in, conv0_w_out, conv0_w_loop, conv0_w_rel, conv0_loop_rel,
           conv0_bias, conv0_gamma, conv0_beta,
           conv1_w_in, conv1_w_out, conv1_w_loop, conv1_w_rel, conv1_loop_rel,
           conv1_bias, conv1_gamma, conv1_beta):
    del quals
    ei0 = edge_index[0].astype(jnp.int32)
    ei1 = edge_index[1].astype(jnp.int32)
    et = edge_type.astype(jnp.int32)

    pad = EPW * NS - E
    z_pad = jnp.zeros((pad,), jnp.int32)
    t_pad = jnp.full((pad,), TRASH, jnp.int32)
    c_pad = jnp.full((pad,), GREL_PAD, jnp.int32)
    srcs = jnp.concatenate([ei0, z_pad, ei1, z_pad]).reshape(2 * NS * NCH, 128)
    dsts = jnp.concatenate([ei1, t_pad, ei0, t_pad]).reshape(2 * NS * NCH, 128)
    cols = jnp.concatenate([et, c_pad, et + 256, c_pad]).reshape(2 * NS * NCH, 128)

    zeros_a = jnp.zeros((ROWS_A, D), jnp.float32)
    ones_tab = jnp.ones((512, D), jnp.float32)

    ra0p = jnp.concatenate(
        [init_rel, conv0_loop_rel, jnp.zeros((6, D), jnp.float32)], axis=0)
    zrow56 = jnp.zeros((56, D), jnp.float32)
    grel = jnp.concatenate([ra0p[:200], zrow56, ra0p[200:400], zrow56], axis=0)
    featb8 = _pack8(feat_b)
    l18 = _pack8(conv1_loop_rel[0])

    h0, ra1p, routp = _tc_feat(x, feat_W, featb8, ra0p, conv0_w_rel,
                               conv1_w_rel, l18)

    r0 = _sc_accum(grel, cols, dsts, zeros_a)
    cnt = _sc_accum(ones_tab, cols, dsts, zeros_a)
    r1 = _tc_relstep(r0, conv0_w_rel)

    rins = (r0[:N], r1[:N])
    routs = (r0[ROWS_A:ROWS_A + N], r1[ROWS_A:ROWS_A + N])
    cin = cnt[:N]
    cout = cnt[ROWS_A:ROWS_A + N]

    h = h0
    params = [
        (conv0_w_in, conv0_w_out, conv0_w_loop, conv0_loop_rel, conv0_bias,
         conv0_gamma, conv0_beta),
        (conv1_w_in, conv1_w_out, conv1_w_loop, conv1_loop_rel,
         conv1_bias, conv1_gamma, conv1_beta),
    ]
    for li, (w_in, w_out, w_loop, loop_rel, bias, gamma, beta) in enumerate(params):
        a = _sc_accum(h, srcs, dsts, zeros_a)
        ain = a[:N]
        aout = a[ROWS_A:ROWS_A + N]
        sm = _pack8(loop_rel[0], bias, gamma, beta)
        pre, stats = _tc_combine(ain, aout, rins[li], routs[li], cin, cout, h,
                                 w_in, w_out, w_loop, sm)
        h = _tc_bn(pre, stats, sm)

    return (h, routp[:401])


# speedup vs baseline: 2.8819x; 1.3456x over previous
import jax
import jax.numpy as jnp
from jax import lax
from jax.experimental import pallas as pl
from jax.experimental.pallas import tpu as pltpu
from jax.experimental.pallas import tpu_sc as plsc

N = 10000
E = 320000
FD = 256
D = 128
NR = 200

NC = 2
NS = 16

EPW = 20480
NCH = EPW // 128
ROWS_A = 10112
SLAB = ROWS_A // NS
TRASH = 10016
GREL_PAD = 500


GRP = 40


def _sc_accum_kernel(h, srcs, dsts, zeros_a, out, acc,
                     sidx, didx, gbuf0, gbuf1, sem0, sem1):
    c = lax.axis_index("c")
    s = lax.axis_index("s")

    pltpu.sync_copy(zeros_a.at[pl.ds(s * SLAB, SLAB)],
                    acc.at[pl.ds(s * SLAB, SLAB)])
    rbase = (c * NS + s) * NCH
    plsc.subcore_barrier()

    for grp in range(NCH // GRP):
        pltpu.sync_copy(srcs.at[pl.ds(rbase + grp * GRP, GRP)], sidx)
        pltpu.sync_copy(dsts.at[pl.ds(rbase + grp * GRP, GRP)], didx)
        pltpu.async_copy(h.at[sidx.at[0]], gbuf0, sem0)
        pltpu.async_copy(h.at[sidx.at[1]], gbuf1, sem1)

        def body(i, carry):
            k0 = 2 * i
            k1 = 2 * i + 1
            pltpu.make_async_copy(h.at[sidx.at[k0]], gbuf0, sem0).wait()
            pltpu.sync_copy(gbuf0, acc.at[didx.at[k0]], add=True)

            @pl.when(k0 + 2 < GRP)
            def _():
                pltpu.async_copy(h.at[sidx.at[k0 + 2]], gbuf0, sem0)

            pltpu.make_async_copy(h.at[sidx.at[k1]], gbuf1, sem1).wait()
            pltpu.sync_copy(gbuf1, acc.at[didx.at[k1]], add=True)

            @pl.when(k1 + 2 < GRP)
            def _():
                pltpu.async_copy(h.at[sidx.at[k1 + 2]], gbuf1, sem1)

            return carry

        lax.fori_loop(0, GRP // 2, body, 0)
    plsc.subcore_barrier()

    ob = c * ROWS_A + s * SLAB
    pltpu.sync_copy(acc.at[pl.ds(s * SLAB, SLAB)], out.at[pl.ds(ob, SLAB)])


def _sc_cnt_kernel(dsts, ones_tab, zeros_a, out, acc, didx, onesb, sem):
    c = lax.axis_index("c")
    s = lax.axis_index("s")

    pltpu.sync_copy(zeros_a.at[pl.ds(s * SLAB, SLAB)],
                    acc.at[pl.ds(s * SLAB, SLAB)])
    pltpu.sync_copy(ones_tab.at[pl.ds(0, 128)], onesb)
    rbase = (c * NS + s) * NCH
    plsc.subcore_barrier()

    half = NCH // 2

    def fire(j, carry):
        pltpu.async_copy(onesb, acc.at[didx.at[j]], sem, add=True)
        return carry

    def drain(j, carry):
        pltpu.make_async_copy(onesb, acc.at[didx.at[0]], sem).wait()
        return carry

    for hh in range(2):
        pltpu.sync_copy(dsts.at[pl.ds(rbase + hh * half, half)], didx)
        lax.fori_loop(0, half, fire, 0)
        lax.fori_loop(0, half, drain, 0)
    plsc.subcore_barrier()

    ob = c * ROWS_A + s * SLAB
    pltpu.sync_copy(acc.at[pl.ds(s * SLAB, SLAB)], out.at[pl.ds(ob, SLAB)])


def _sc_accum(h, srcs, dsts, zeros_a):
    return pl.kernel(
        _sc_accum_kernel,
        out_type=jax.ShapeDtypeStruct((2 * ROWS_A, D), jnp.float32),
        mesh=plsc.VectorSubcoreMesh(core_axis_name="c", subcore_axis_name="s"),
        scratch_types=[
            pltpu.VMEM_SHARED((ROWS_A, D), jnp.float32),
            pltpu.VMEM((GRP, 128), jnp.int32),
            pltpu.VMEM((GRP, 128), jnp.int32),
            pltpu.VMEM((128, D), jnp.float32),
            pltpu.VMEM((128, D), jnp.float32),
            pltpu.SemaphoreType.DMA,
            pltpu.SemaphoreType.DMA,
        ],
    )(h, srcs, dsts, zeros_a)


def _sc_cnt(dsts, ones_tab, zeros_a):
    return pl.kernel(
        _sc_cnt_kernel,
        out_type=jax.ShapeDtypeStruct((2 * ROWS_A, D), jnp.float32),
        mesh=plsc.VectorSubcoreMesh(core_axis_name="c", subcore_axis_name="s"),
        scratch_types=[
            pltpu.VMEM_SHARED((ROWS_A, D), jnp.float32),
            pltpu.VMEM((NCH // 2, 128), jnp.int32),
            pltpu.VMEM((128, D), jnp.float32),
            pltpu.SemaphoreType.DMA,
        ],
    )(dsts, ones_tab, zeros_a)


def _feat_kernel(x_ref, w_ref, b_ref, ra0_ref, wr0_ref, wr1_ref, l1_ref,
                 h_ref, ra1_ref, rout_ref):
    h_ref[...] = jnp.dot(x_ref[...], w_ref[...],
                         preferred_element_type=jnp.float32) + b_ref[0:1, :]
    ra1 = jnp.dot(ra0_ref[...], wr0_ref[...], preferred_element_type=jnp.float32)
    rows = lax.broadcasted_iota(jnp.int32, ra1.shape, 0)
    ra1 = jnp.where(rows == 401, l1_ref[0:1, :], ra1)
    ra1_ref[...] = ra1
    rout_ref[...] = jnp.dot(ra1, wr1_ref[...], preferred_element_type=jnp.float32)


def _combine_kernel(ain_ref, aout_ref, rin_ref, rout_ref, cin_ref, cout_ref,
                    h_ref, win_ref, wout_ref, wloop_ref,
                    sm_ref, pre_ref, stats_ref, acc):
    i = pl.program_id(0)

    @pl.when(i == 0)
    def _():
        acc[...] = jnp.zeros_like(acc)

    ni = jnp.maximum(cin_ref[:, 0:1], 1.0)
    no = jnp.maximum(cout_ref[:, 0:1], 1.0)
    in_res = jnp.dot((ain_ref[...] - rin_ref[...]) / ni, win_ref[...],
                     preferred_element_type=jnp.float32)
    out_res = jnp.dot((aout_ref[...] - rout_ref[...]) / no, wout_ref[...],
                      preferred_element_type=jnp.float32)
    loop_res = jnp.dot(h_ref[...] - sm_ref[0:1, :], wloop_ref[...],
                       preferred_element_type=jnp.float32)
    pre = (in_res + out_res + loop_res) / 3.0 + sm_ref[1:2, :]
    pre_ref[...] = pre
    acc[0:1, :] += jnp.sum(pre, axis=0, keepdims=True)
    acc[1:2, :] += jnp.sum(pre * pre, axis=0, keepdims=True)

    @pl.when(i == pl.num_programs(0) - 1)
    def _():
        stats_ref[...] = acc[...]


def _relstep_kernel(r_ref, w_ref, o_ref):
    o_ref[...] = jnp.dot(r_ref[...], w_ref[...],
                         preferred_element_type=jnp.float32)


def _bn_kernel(pre_ref, stats_ref, sm_ref, h_ref):
    mu = stats_ref[0:1, :] / float(N)
    ex2 = stats_ref[1:2, :] / float(N)
    var = ex2 - mu * mu
    inv = lax.rsqrt(var + 1e-5)
    h = (pre_ref[...] - mu) * inv * sm_ref[2:3, :] + sm_ref[3:4, :]
    h_ref[...] = jnp.maximum(h, 0.0)


def _tc_feat(x, feat_W, featb8, ra0p, wr0, wr1, l18):
    return pl.pallas_call(
        _feat_kernel,
        out_shape=(
            jax.ShapeDtypeStruct((N, D), jnp.float32),
            jax.ShapeDtypeStruct((408, D), jnp.float32),
            jax.ShapeDtypeStruct((408, D), jnp.float32),
        ),
    )(x, feat_W, featb8, ra0p, wr0, wr1, l18)


def _tc_combine(ain, aout, rin, rout, cin, cout, h, win, wout, wloop, sm):
    nb = 10
    blk = N // nb
    row = lambda i: (i, 0)
    full = lambda i: (0, 0)
    return pl.pallas_call(
        _combine_kernel,
        grid=(nb,),
        in_specs=[
            pl.BlockSpec((blk, D), row),
            pl.BlockSpec((blk, D), row),
            pl.BlockSpec((blk, D), row),
            pl.BlockSpec((blk, D), row),
            pl.BlockSpec((blk, D), row),
            pl.BlockSpec((blk, D), row),
            pl.BlockSpec((blk, D), row),
            pl.BlockSpec((D, D), full),
            pl.BlockSpec((D, D), full),
            pl.BlockSpec((D, D), full),
            pl.BlockSpec((8, D), full),
        ],
        out_specs=[
            pl.BlockSpec((blk, D), row),
            pl.BlockSpec((8, D), full),
        ],
        out_shape=(
            jax.ShapeDtypeStruct((N, D), jnp.float32),
            jax.ShapeDtypeStruct((8, D), jnp.float32),
        ),
        scratch_shapes=[pltpu.VMEM((8, D), jnp.float32)],
    )(ain, aout, rin, rout, cin, cout, h, win, wout, wloop, sm)


def _tc_relstep(r2, w):
    return pl.pallas_call(
        _relstep_kernel,
        out_shape=jax.ShapeDtypeStruct((2 * ROWS_A, D), jnp.float32),
    )(r2, w)


def _tc_bn(pre, stats, sm):
    return pl.pallas_call(
        _bn_kernel,
        out_shape=jax.ShapeDtypeStruct((N, D), jnp.float32),
    )(pre, stats, sm)


def _pack8(*rows):
    out = [jnp.reshape(r, (1, D)) for r in rows]
    out += [jnp.zeros((1, D), jnp.float32)] * (8 - len(out))
    return jnp.concatenate(out, axis=0)


def kernel(x, edge_index, edge_type, quals, feat_W, feat_b, init_rel,
           conv0_w_in, conv0_w_out, conv0_w_loop, conv0_w_rel, conv0_loop_rel,
           conv0_bias, conv0_gamma, conv0_beta,
           conv1_w_in, conv1_w_out, conv1_w_loop, conv1_w_rel, conv1_loop_rel,
           conv1_bias, conv1_gamma, conv1_beta):
    del quals
    ei0 = edge_index[0].astype(jnp.int32)
    ei1 = edge_index[1].astype(jnp.int32)
    et = edge_type.astype(jnp.int32)

    pad = EPW * NS - E
    z_pad = jnp.zeros((pad,), jnp.int32)
    t_pad = jnp.full((pad,), TRASH, jnp.int32)
    c_pad = jnp.full((pad,), GREL_PAD, jnp.int32)
    srcs = jnp.concatenate([ei0, z_pad, ei1, z_pad]).reshape(2 * NS * NCH, 128)
    dsts = jnp.concatenate([ei1, t_pad, ei0, t_pad]).reshape(2 * NS * NCH, 128)
    cols = jnp.concatenate([et, c_pad, et + 256, c_pad]).reshape(2 * NS * NCH, 128)

    zeros_a = jnp.zeros((ROWS_A, D), jnp.float32)
    ones_tab = jnp.ones((512, D), jnp.float32)

    ra0p = jnp.concatenate(
        [init_rel, conv0_loop_rel, jnp.zeros((6, D), jnp.float32)], axis=0)
    zrow56 = jnp.zeros((56, D), jnp.float32)
    grel = jnp.concatenate([ra0p[:200], zrow56, ra0p[200:400], zrow56], axis=0)
    featb8 = _pack8(feat_b)
    l18 = _pack8(conv1_loop_rel[0])

    h0, ra1p, routp = _tc_feat(x, feat_W, featb8, ra0p, conv0_w_rel,
                               conv1_w_rel, l18)

    r0 = _sc_accum(grel, cols, dsts, zeros_a)
    cnt = _sc_cnt(dsts, ones_tab, zeros_a)
    r1 = _tc_relstep(r0, conv0_w_rel)

    rins = (r0[:N], r1[:N])
    routs = (r0[ROWS_A:ROWS_A + N], r1[ROWS_A:ROWS_A + N])
    cin = cnt[:N]
    cout = cnt[ROWS_A:ROWS_A + N]

    h = h0
    params = [
        (conv0_w_in, conv0_w_out, conv0_w_loop, conv0_loop_rel, conv0_bias,
         conv0_gamma, conv0_beta),
        (conv1_w_in, conv1_w_out, conv1_w_loop, conv1_loop_rel,
         conv1_bias, conv1_gamma, conv1_beta),
    ]
    for li, (w_in, w_out, w_loop, loop_rel, bias, gamma, beta) in enumerate(params):
        a = _sc_accum(h, srcs, dsts, zeros_a)
        ain = a[:N]
        aout = a[ROWS_A:ROWS_A + N]
        sm = _pack8(loop_rel[0], bias, gamma, beta)
        pre, stats = _tc_combine(ain, aout, rins[li], routs[li], cin, cout, h,
                                 w_in, w_out, w_loop, sm)
        h = _tc_bn(pre, stats, sm)

    return (h, routp[:401])


# speedup vs baseline: 3.6181x; 1.2555x over previous
import jax
import jax.numpy as jnp
from jax import lax
from jax.experimental import pallas as pl
from jax.experimental.pallas import tpu as pltpu
from jax.experimental.pallas import tpu_sc as plsc

N = 10000
E = 320000
FD = 256
D = 128
NR = 200

NC = 2
NS = 16

EPW = 20480
NCH = EPW // 128
ROWS_A = 10112
SLAB = ROWS_A // NS
TRASH = 10016
GREL_PAD = 500


GRP = 40


def _sc_accum_kernel(h, srcs, dsts, zeros_a, out, acc,
                     sidx, didx, gbuf0, gbuf1, sem0, sem1):
    c = lax.axis_index("c")
    s = lax.axis_index("s")

    pltpu.sync_copy(zeros_a.at[pl.ds(s * SLAB, SLAB)],
                    acc.at[pl.ds(s * SLAB, SLAB)])
    rbase = (c * NS + s) * NCH
    plsc.subcore_barrier()

    for grp in range(NCH // GRP):
        pltpu.sync_copy(srcs.at[pl.ds(rbase + grp * GRP, GRP)], sidx)
        pltpu.sync_copy(dsts.at[pl.ds(rbase + grp * GRP, GRP)], didx)
        pltpu.async_copy(h.at[sidx.at[0]], gbuf0, sem0)
        pltpu.async_copy(h.at[sidx.at[1]], gbuf1, sem1)

        def body(i, carry):
            k0 = 2 * i
            k1 = 2 * i + 1
            pltpu.make_async_copy(h.at[sidx.at[k0]], gbuf0, sem0).wait()
            pltpu.sync_copy(gbuf0, acc.at[didx.at[k0]], add=True)

            @pl.when(k0 + 2 < GRP)
            def _():
                pltpu.async_copy(h.at[sidx.at[k0 + 2]], gbuf0, sem0)

            pltpu.make_async_copy(h.at[sidx.at[k1]], gbuf1, sem1).wait()
            pltpu.sync_copy(gbuf1, acc.at[didx.at[k1]], add=True)

            @pl.when(k1 + 2 < GRP)
            def _():
                pltpu.async_copy(h.at[sidx.at[k1 + 2]], gbuf1, sem1)

            return carry

        lax.fori_loop(0, GRP // 2, body, 0)
    plsc.subcore_barrier()

    ob = c * ROWS_A + s * SLAB
    pltpu.sync_copy(acc.at[pl.ds(s * SLAB, SLAB)], out.at[pl.ds(ob, SLAB)])


def _sc_rel_kernel(grel, srcs, dsts, zeros_a, out, acc, tab,
                   sidx, didx, gbuf0, gbuf1, sem0, sem1):
    c = lax.axis_index("c")
    s = lax.axis_index("s")

    pltpu.sync_copy(zeros_a.at[pl.ds(s * SLAB, SLAB)],
                    acc.at[pl.ds(s * SLAB, SLAB)])

    @pl.when(s == 0)
    def _():
        pltpu.sync_copy(grel, tab)

    rbase = (c * NS + s) * NCH
    plsc.subcore_barrier()

    for grp in range(NCH // GRP):
        pltpu.sync_copy(srcs.at[pl.ds(rbase + grp * GRP, GRP)], sidx)
        pltpu.sync_copy(dsts.at[pl.ds(rbase + grp * GRP, GRP)], didx)
        pltpu.async_copy(tab.at[sidx.at[0]], gbuf0, sem0)
        pltpu.async_copy(tab.at[sidx.at[1]], gbuf1, sem1)

        def body(i, carry):
            k0 = 2 * i
            k1 = 2 * i + 1
            pltpu.make_async_copy(tab.at[sidx.at[k0]], gbuf0, sem0).wait()
            pltpu.sync_copy(gbuf0, acc.at[didx.at[k0]], add=True)

            @pl.when(k0 + 2 < GRP)
            def _():
                pltpu.async_copy(tab.at[sidx.at[k0 + 2]], gbuf0, sem0)

            pltpu.make_async_copy(tab.at[sidx.at[k1]], gbuf1, sem1).wait()
            pltpu.sync_copy(gbuf1, acc.at[didx.at[k1]], add=True)

            @pl.when(k1 + 2 < GRP)
            def _():
                pltpu.async_copy(tab.at[sidx.at[k1 + 2]], gbuf1, sem1)

            return carry

        lax.fori_loop(0, GRP // 2, body, 0)
    plsc.subcore_barrier()

    ob = c * ROWS_A + s * SLAB
    pltpu.sync_copy(acc.at[pl.ds(s * SLAB, SLAB)], out.at[pl.ds(ob, SLAB)])


def _sc_cnt_kernel(dsts, ones_tab, zeros_a, out, acc, didx, onesb, sem):
    c = lax.axis_index("c")
    s = lax.axis_index("s")

    pltpu.sync_copy(zeros_a.at[pl.ds(s * SLAB, SLAB)],
                    acc.at[pl.ds(s * SLAB, SLAB)])
    pltpu.sync_copy(ones_tab.at[pl.ds(0, 128)], onesb)
    rbase = (c * NS + s) * NCH
    plsc.subcore_barrier()

    half = NCH // 2

    def fire(j, carry):
        pltpu.async_copy(onesb, acc.at[didx.at[j]], sem, add=True)
        return carry

    def drain(j, carry):
        pltpu.make_async_copy(onesb, acc.at[didx.at[0]], sem).wait()
        return carry

    for hh in range(2):
        pltpu.sync_copy(dsts.at[pl.ds(rbase + hh * half, half)], didx)
        lax.fori_loop(0, half, fire, 0)
        lax.fori_loop(0, half, drain, 0)
    plsc.subcore_barrier()

    ob = c * ROWS_A + s * SLAB
    pltpu.sync_copy(acc.at[pl.ds(s * SLAB, SLAB)], out.at[pl.ds(ob, SLAB)])


def _sc_accum(h, srcs, dsts, zeros_a):
    return pl.kernel(
        _sc_accum_kernel,
        out_type=jax.ShapeDtypeStruct((2 * ROWS_A, D), jnp.float32),
        mesh=plsc.VectorSubcoreMesh(core_axis_name="c", subcore_axis_name="s"),
        scratch_types=[
            pltpu.VMEM_SHARED((ROWS_A, D), jnp.float32),
            pltpu.VMEM((GRP, 128), jnp.int32),
            pltpu.VMEM((GRP, 128), jnp.int32),
            pltpu.VMEM((128, D), jnp.float32),
            pltpu.VMEM((128, D), jnp.float32),
            pltpu.SemaphoreType.DMA,
            pltpu.SemaphoreType.DMA,
        ],
    )(h, srcs, dsts, zeros_a)


def _sc_rel(grel, cols, dsts, zeros_a):
    return pl.kernel(
        _sc_rel_kernel,
        out_type=jax.ShapeDtypeStruct((2 * ROWS_A, D), jnp.float32),
        mesh=plsc.VectorSubcoreMesh(core_axis_name="c", subcore_axis_name="s"),
        scratch_types=[
            pltpu.VMEM_SHARED((ROWS_A, D), jnp.float32),
            pltpu.VMEM_SHARED((512, D), jnp.float32),
            pltpu.VMEM((GRP, 128), jnp.int32),
            pltpu.VMEM((GRP, 128), jnp.int32),
            pltpu.VMEM((128, D), jnp.float32),
            pltpu.VMEM((128, D), jnp.float32),
            pltpu.SemaphoreType.DMA,
            pltpu.SemaphoreType.DMA,
        ],
    )(grel, cols, dsts, zeros_a)


def _sc_cnt(dsts, ones_tab, zeros_a):
    return pl.kernel(
        _sc_cnt_kernel,
        out_type=jax.ShapeDtypeStruct((2 * ROWS_A, D), jnp.float32),
        mesh=plsc.VectorSubcoreMesh(core_axis_name="c", subcore_axis_name="s"),
        scratch_types=[
            pltpu.VMEM_SHARED((ROWS_A, D), jnp.float32),
            pltpu.VMEM((NCH // 2, 128), jnp.int32),
            pltpu.VMEM((128, D), jnp.float32),
            pltpu.SemaphoreType.DMA,
        ],
    )(dsts, ones_tab, zeros_a)


def _feat_kernel(x_ref, w_ref, b_ref, ra0_ref, wr0_ref, wr1_ref, l1_ref,
                 h_ref, ra1_ref, rout_ref):
    h_ref[...] = jnp.dot(x_ref[...], w_ref[...],
                         preferred_element_type=jnp.float32) + b_ref[0:1, :]
    ra1 = jnp.dot(ra0_ref[...], wr0_ref[...], preferred_element_type=jnp.float32)
    rows = lax.broadcasted_iota(jnp.int32, ra1.shape, 0)
    ra1 = jnp.where(rows == 401, l1_ref[0:1, :], ra1)
    ra1_ref[...] = ra1
    rout_ref[...] = jnp.dot(ra1, wr1_ref[...], preferred_element_type=jnp.float32)


def _combine_kernel(ain_ref, aout_ref, rin_ref, rout_ref, cin_ref, cout_ref,
                    h_ref, win_ref, wout_ref, wloop_ref,
                    sm_ref, pre_ref, stats_ref, acc):
    i = pl.program_id(0)

    @pl.when(i == 0)
    def _():
        acc[...] = jnp.zeros_like(acc)

    ni = jnp.maximum(cin_ref[:, 0:1], 1.0)
    no = jnp.maximum(cout_ref[:, 0:1], 1.0)
    in_res = jnp.dot((ain_ref[...] - rin_ref[...]) / ni, win_ref[...],
                     preferred_element_type=jnp.float32)
    out_res = jnp.dot((aout_ref[...] - rout_ref[...]) / no, wout_ref[...],
                      preferred_element_type=jnp.float32)
    loop_res = jnp.dot(h_ref[...] - sm_ref[0:1, :], wloop_ref[...],
                       preferred_element_type=jnp.float32)
    pre = (in_res + out_res + loop_res) / 3.0 + sm_ref[1:2, :]
    pre_ref[...] = pre
    acc[0:1, :] += jnp.sum(pre, axis=0, keepdims=True)
    acc[1:2, :] += jnp.sum(pre * pre, axis=0, keepdims=True)

    @pl.when(i == pl.num_programs(0) - 1)
    def _():
        stats_ref[...] = acc[...]


def _relstep_kernel(r_ref, w_ref, o_ref):
    o_ref[...] = jnp.dot(r_ref[...], w_ref[...],
                         preferred_element_type=jnp.float32)


def _bn_kernel(pre_ref, stats_ref, sm_ref, h_ref):
    mu = stats_ref[0:1, :] / float(N)
    ex2 = stats_ref[1:2, :] / float(N)
    var = ex2 - mu * mu
    inv = lax.rsqrt(var + 1e-5)
    h = (pre_ref[...] - mu) * inv * sm_ref[2:3, :] + sm_ref[3:4, :]
    h_ref[...] = jnp.maximum(h, 0.0)


def _tc_feat(x, feat_W, featb8, ra0p, wr0, wr1, l18):
    return pl.pallas_call(
        _feat_kernel,
        out_shape=(
            jax.ShapeDtypeStruct((N, D), jnp.float32),
            jax.ShapeDtypeStruct((408, D), jnp.float32),
            jax.ShapeDtypeStruct((408, D), jnp.float32),
        ),
    )(x, feat_W, featb8, ra0p, wr0, wr1, l18)


def _tc_combine(ain, aout, rin, rout, cin, cout, h, win, wout, wloop, sm):
    nb = 10
    blk = N // nb
    row = lambda i: (i, 0)
    full = lambda i: (0, 0)
    return pl.pallas_call(
        _combine_kernel,
        grid=(nb,),
        in_specs=[
            pl.BlockSpec((blk, D), row),
            pl.BlockSpec((blk, D), row),
            pl.BlockSpec((blk, D), row),
            pl.BlockSpec((blk, D), row),
            pl.BlockSpec((blk, D), row),
            pl.BlockSpec((blk, D), row),
            pl.BlockSpec((blk, D), row),
            pl.BlockSpec((D, D), full),
            pl.BlockSpec((D, D), full),
            pl.BlockSpec((D, D), full),
            pl.BlockSpec((8, D), full),
        ],
        out_specs=[
            pl.BlockSpec((blk, D), row),
            pl.BlockSpec((8, D), full),
        ],
        out_shape=(
            jax.ShapeDtypeStruct((N, D), jnp.float32),
            jax.ShapeDtypeStruct((8, D), jnp.float32),
        ),
        scratch_shapes=[pltpu.VMEM((8, D), jnp.float32)],
    )(ain, aout, rin, rout, cin, cout, h, win, wout, wloop, sm)


def _tc_relstep(r2, w):
    return pl.pallas_call(
        _relstep_kernel,
        out_shape=jax.ShapeDtypeStruct((2 * ROWS_A, D), jnp.float32),
    )(r2, w)


def _tc_bn(pre, stats, sm):
    return pl.pallas_call(
        _bn_kernel,
        out_shape=jax.ShapeDtypeStruct((N, D), jnp.float32),
    )(pre, stats, sm)


def _pack8(*rows):
    out = [jnp.reshape(r, (1, D)) for r in rows]
    out += [jnp.zeros((1, D), jnp.float32)] * (8 - len(out))
    return jnp.concatenate(out, axis=0)


def kernel(x, edge_index, edge_type, quals, feat_W, feat_b, init_rel,
           conv0_w_in, conv0_w_out, conv0_w_loop, conv0_w_rel, conv0_loop_rel,
           conv0_bias, conv0_gamma, conv0_beta,
           conv1_w_in, conv1_w_out, conv1_w_loop, conv1_w_rel, conv1_loop_rel,
           conv1_bias, conv1_gamma, conv1_beta):
    del quals
    ei0 = edge_index[0].astype(jnp.int32)
    ei1 = edge_index[1].astype(jnp.int32)
    et = edge_type.astype(jnp.int32)

    pad = EPW * NS - E
    z_pad = jnp.zeros((pad,), jnp.int32)
    t_pad = jnp.full((pad,), TRASH, jnp.int32)
    c_pad = jnp.full((pad,), GREL_PAD, jnp.int32)
    srcs = jnp.concatenate([ei0, z_pad, ei1, z_pad]).reshape(2 * NS * NCH, 128)
    dsts = jnp.concatenate([ei1, t_pad, ei0, t_pad]).reshape(2 * NS * NCH, 128)
    cols = jnp.concatenate([et, c_pad, et + 256, c_pad]).reshape(2 * NS * NCH, 128)

    zeros_a = jnp.zeros((ROWS_A, D), jnp.float32)
    ones_tab = jnp.ones((512, D), jnp.float32)

    ra0p = jnp.concatenate(
        [init_rel, conv0_loop_rel, jnp.zeros((6, D), jnp.float32)], axis=0)
    zrow56 = jnp.zeros((56, D), jnp.float32)
    grel = jnp.concatenate([ra0p[:200], zrow56, ra0p[200:400], zrow56], axis=0)
    featb8 = _pack8(feat_b)
    l18 = _pack8(conv1_loop_rel[0])

    h0, ra1p, routp = _tc_feat(x, feat_W, featb8, ra0p, conv0_w_rel,
                               conv1_w_rel, l18)

    r0 = _sc_rel(grel, cols, dsts, zeros_a)
    cnt = _sc_cnt(dsts, ones_tab, zeros_a)
    r1 = _tc_relstep(r0, conv0_w_rel)

    rins = (r0[:N], r1[:N])
    routs = (r0[ROWS_A:ROWS_A + N], r1[ROWS_A:ROWS_A + N])
    cin = cnt[:N]
    cout = cnt[ROWS_A:ROWS_A + N]

    h = h0
    params = [
        (conv0_w_in, conv0_w_out, conv0_w_loop, conv0_loop_rel, conv0_bias,
         conv0_gamma, conv0_beta),
        (conv1_w_in, conv1_w_out, conv1_w_loop, conv1_loop_rel,
         conv1_bias, conv1_gamma, conv1_beta),
    ]
    for li, (w_in, w_out, w_loop, loop_rel, bias, gamma, beta) in enumerate(params):
        a = _sc_accum(h, srcs, dsts, zeros_a)
        ain = a[:N]
        aout = a[ROWS_A:ROWS_A + N]
        sm = _pack8(loop_rel[0], bias, gamma, beta)
        pre, stats = _tc_combine(ain, aout, rins[li], routs[li], cin, cout, h,
                                 w_in, w_out, w_loop, sm)
        h = _tc_bn(pre, stats, sm)

    return (h, routp[:401])


# speedup vs baseline: 3.6217x; 1.0010x over previous
import jax
import jax.numpy as jnp
from jax import lax
from jax.experimental import pallas as pl
from jax.experimental.pallas import tpu as pltpu
from jax.experimental.pallas import tpu_sc as plsc

N = 10000
E = 320000
FD = 256
D = 128
NR = 200

NC = 2
NS = 16

EPW = 20480
NCH = EPW // 128
ROWS_A = 10112
SLAB = ROWS_A // NS
TRASH = 10016
GREL_PAD = 500


GRP = 40


def _sc_accum_kernel(h, srcs, dsts, zeros_a, out, acc,
                     sidx, didx, gbuf0, gbuf1, sem0, sem1):
    c = lax.axis_index("c")
    s = lax.axis_index("s")

    pltpu.sync_copy(zeros_a.at[pl.ds(s * SLAB, SLAB)],
                    acc.at[pl.ds(s * SLAB, SLAB)])
    rbase = (c * NS + s) * NCH
    plsc.subcore_barrier()

    for grp in range(NCH // GRP):
        pltpu.sync_copy(srcs.at[pl.ds(rbase + grp * GRP, GRP)], sidx)
        pltpu.sync_copy(dsts.at[pl.ds(rbase + grp * GRP, GRP)], didx)
        pltpu.async_copy(h.at[sidx.at[0]], gbuf0, sem0)
        pltpu.async_copy(h.at[sidx.at[1]], gbuf1, sem1, priority=1)

        def body(i, carry):
            k0 = 2 * i
            k1 = 2 * i + 1
            pltpu.make_async_copy(h.at[sidx.at[k0]], gbuf0, sem0).wait()
            pltpu.sync_copy(gbuf0, acc.at[didx.at[k0]], add=True)

            @pl.when(k0 + 2 < GRP)
            def _():
                pltpu.async_copy(h.at[sidx.at[k0 + 2]], gbuf0, sem0)

            pltpu.make_async_copy(h.at[sidx.at[k1]], gbuf1, sem1).wait()
            pltpu.sync_copy(gbuf1, acc.at[didx.at[k1]], add=True)

            @pl.when(k1 + 2 < GRP)
            def _():
                pltpu.async_copy(h.at[sidx.at[k1 + 2]], gbuf1, sem1, priority=1)

            return carry

        lax.fori_loop(0, GRP // 2, body, 0)
    plsc.subcore_barrier()

    ob = c * ROWS_A + s * SLAB
    pltpu.sync_copy(acc.at[pl.ds(s * SLAB, SLAB)], out.at[pl.ds(ob, SLAB)])


def _sc_rel_kernel(grel, srcs, dsts, zeros_a, out, acc, tab,
                   sidx, didx, gbuf0, gbuf1, sem0, sem1):
    c = lax.axis_index("c")
    s = lax.axis_index("s")

    pltpu.sync_copy(zeros_a.at[pl.ds(s * SLAB, SLAB)],
                    acc.at[pl.ds(s * SLAB, SLAB)])

    @pl.when(s == 0)
    def _():
        pltpu.sync_copy(grel, tab)

    rbase = (c * NS + s) * NCH
    plsc.subcore_barrier()

    for grp in range(NCH // GRP):
        pltpu.sync_copy(srcs.at[pl.ds(rbase + grp * GRP, GRP)], sidx)
        pltpu.sync_copy(dsts.at[pl.ds(rbase + grp * GRP, GRP)], didx)
        pltpu.async_copy(tab.at[sidx.at[0]], gbuf0, sem0)
        pltpu.async_copy(tab.at[sidx.at[1]], gbuf1, sem1)

        def body(i, carry):
            k0 = 2 * i
            k1 = 2 * i + 1
            pltpu.make_async_copy(tab.at[sidx.at[k0]], gbuf0, sem0).wait()
            pltpu.sync_copy(gbuf0, acc.at[didx.at[k0]], add=True)

            @pl.when(k0 + 2 < GRP)
            def _():
                pltpu.async_copy(tab.at[sidx.at[k0 + 2]], gbuf0, sem0)

            pltpu.make_async_copy(tab.at[sidx.at[k1]], gbuf1, sem1).wait()
            pltpu.sync_copy(gbuf1, acc.at[didx.at[k1]], add=True)

            @pl.when(k1 + 2 < GRP)
            def _():
                pltpu.async_copy(tab.at[sidx.at[k1 + 2]], gbuf1, sem1)

            return carry

        lax.fori_loop(0, GRP // 2, body, 0)
    plsc.subcore_barrier()

    ob = c * ROWS_A + s * SLAB
    pltpu.sync_copy(acc.at[pl.ds(s * SLAB, SLAB)], out.at[pl.ds(ob, SLAB)])


def _sc_cnt_kernel(dsts, ones_tab, zeros_a, out, acc, didx, onesb, sem):
    c = lax.axis_index("c")
    s = lax.axis_index("s")

    pltpu.sync_copy(zeros_a.at[pl.ds(s * SLAB, SLAB)],
                    acc.at[pl.ds(s * SLAB, SLAB)])
    pltpu.sync_copy(ones_tab.at[pl.ds(0, 128)], onesb)
    rbase = (c * NS + s) * NCH
    plsc.subcore_barrier()

    half = NCH // 2

    def fire(j, carry):
        pltpu.async_copy(onesb, acc.at[didx.at[j]], sem, add=True)
        return carry

    def drain(j, carry):
        pltpu.make_async_copy(onesb, acc.at[didx.at[0]], sem).wait()
        return carry

    for hh in range(2):
        pltpu.sync_copy(dsts.at[pl.ds(rbase + hh * half, half)], didx)
        lax.fori_loop(0, half, fire, 0)
        lax.fori_loop(0, half, drain, 0)
    plsc.subcore_barrier()

    ob = c * ROWS_A + s * SLAB
    pltpu.sync_copy(acc.at[pl.ds(s * SLAB, SLAB)], out.at[pl.ds(ob, SLAB)])


def _sc_accum(h, srcs, dsts, zeros_a):
    return pl.kernel(
        _sc_accum_kernel,
        out_type=jax.ShapeDtypeStruct((2 * ROWS_A, D), jnp.float32),
        mesh=plsc.VectorSubcoreMesh(core_axis_name="c", subcore_axis_name="s"),
        scratch_types=[
            pltpu.VMEM_SHARED((ROWS_A, D), jnp.float32),
            pltpu.VMEM((GRP, 128), jnp.int32),
            pltpu.VMEM((GRP, 128), jnp.int32),
            pltpu.VMEM((128, D), jnp.float32),
            pltpu.VMEM((128, D), jnp.float32),
            pltpu.SemaphoreType.DMA,
            pltpu.SemaphoreType.DMA,
        ],
    )(h, srcs, dsts, zeros_a)


def _sc_rel(grel, cols, dsts, zeros_a):
    return pl.kernel(
        _sc_rel_kernel,
        out_type=jax.ShapeDtypeStruct((2 * ROWS_A, D), jnp.float32),
        mesh=plsc.VectorSubcoreMesh(core_axis_name="c", subcore_axis_name="s"),
        scratch_types=[
            pltpu.VMEM_SHARED((ROWS_A, D), jnp.float32),
            pltpu.VMEM_SHARED((512, D), jnp.float32),
            pltpu.VMEM((GRP, 128), jnp.int32),
            pltpu.VMEM((GRP, 128), jnp.int32),
            pltpu.VMEM((128, D), jnp.float32),
            pltpu.VMEM((128, D), jnp.float32),
            pltpu.SemaphoreType.DMA,
            pltpu.SemaphoreType.DMA,
        ],
    )(grel, cols, dsts, zeros_a)


def _sc_cnt(dsts, ones_tab, zeros_a):
    return pl.kernel(
        _sc_cnt_kernel,
        out_type=jax.ShapeDtypeStruct((2 * ROWS_A, D), jnp.float32),
        mesh=plsc.VectorSubcoreMesh(core_axis_name="c", subcore_axis_name="s"),
        scratch_types=[
            pltpu.VMEM_SHARED((ROWS_A, D), jnp.float32),
            pltpu.VMEM((NCH // 2, 128), jnp.int32),
            pltpu.VMEM((128, D), jnp.float32),
            pltpu.SemaphoreType.DMA,
        ],
    )(dsts, ones_tab, zeros_a)


def _feat_kernel(x_ref, w_ref, b_ref, ra0_ref, wr0_ref, wr1_ref, l1_ref,
                 h_ref, ra1_ref, rout_ref):
    h_ref[...] = jnp.dot(x_ref[...], w_ref[...],
                         preferred_element_type=jnp.float32) + b_ref[0:1, :]
    ra1 = jnp.dot(ra0_ref[...], wr0_ref[...], preferred_element_type=jnp.float32)
    rows = lax.broadcasted_iota(jnp.int32, ra1.shape, 0)
    ra1 = jnp.where(rows == 401, l1_ref[0:1, :], ra1)
    ra1_ref[...] = ra1
    rout_ref[...] = jnp.dot(ra1, wr1_ref[...], preferred_element_type=jnp.float32)


def _combine_kernel(ain_ref, aout_ref, rin_ref, rout_ref, cin_ref, cout_ref,
                    h_ref, win_ref, wout_ref, wloop_ref,
                    sm_ref, pre_ref, stats_ref, acc):
    i = pl.program_id(0)

    @pl.when(i == 0)
    def _():
        acc[...] = jnp.zeros_like(acc)

    ni = jnp.maximum(cin_ref[:, 0:1], 1.0)
    no = jnp.maximum(cout_ref[:, 0:1], 1.0)
    in_res = jnp.dot((ain_ref[...] - rin_ref[...]) / ni, win_ref[...],
                     preferred_element_type=jnp.float32)
    out_res = jnp.dot((aout_ref[...] - rout_ref[...]) / no, wout_ref[...],
                      preferred_element_type=jnp.float32)
    loop_res = jnp.dot(h_ref[...] - sm_ref[0:1, :], wloop_ref[...],
                       preferred_element_type=jnp.float32)
    pre = (in_res + out_res + loop_res) / 3.0 + sm_ref[1:2, :]
    pre_ref[...] = pre
    acc[0:1, :] += jnp.sum(pre, axis=0, keepdims=True)
    acc[1:2, :] += jnp.sum(pre * pre, axis=0, keepdims=True)

    @pl.when(i == pl.num_programs(0) - 1)
    def _():
        stats_ref[...] = acc[...]


def _relstep_kernel(r_ref, w_ref, o_ref):
    o_ref[...] = jnp.dot(r_ref[...], w_ref[...],
                         preferred_element_type=jnp.float32)


def _bn_kernel(pre_ref, stats_ref, sm_ref, h_ref):
    mu = stats_ref[0:1, :] / float(N)
    ex2 = stats_ref[1:2, :] / float(N)
    var = ex2 - mu * mu
    inv = lax.rsqrt(var + 1e-5)
    h = (pre_ref[...] - mu) * inv * sm_ref[2:3, :] + sm_ref[3:4, :]
    h_ref[...] = jnp.maximum(h, 0.0)


def _tc_feat(x, feat_W, featb8, ra0p, wr0, wr1, l18):
    return pl.pallas_call(
        _feat_kernel,
        out_shape=(
            jax.ShapeDtypeStruct((N, D), jnp.float32),
            jax.ShapeDtypeStruct((408, D), jnp.float32),
            jax.ShapeDtypeStruct((408, D), jnp.float32),
        ),
    )(x, feat_W, featb8, ra0p, wr0, wr1, l18)


def _tc_combine(ain, aout, rin, rout, cin, cout, h, win, wout, wloop, sm):
    nb = 10
    blk = N // nb
    row = lambda i: (i, 0)
    full = lambda i: (0, 0)
    return pl.pallas_call(
        _combine_kernel,
        grid=(nb,),
        in_specs=[
            pl.BlockSpec((blk, D), row),
            pl.BlockSpec((blk, D), row),
            pl.BlockSpec((blk, D), row),
            pl.BlockSpec((blk, D), row),
            pl.BlockSpec((blk, D), row),
            pl.BlockSpec((blk, D), row),
            pl.BlockSpec((blk, D), row),
            pl.BlockSpec((D, D), full),
            pl.BlockSpec((D, D), full),
            pl.BlockSpec((D, D), full),
            pl.BlockSpec((8, D), full),
        ],
        out_specs=[
            pl.BlockSpec((blk, D), row),
            pl.BlockSpec((8, D), full),
        ],
        out_shape=(
            jax.ShapeDtypeStruct((N, D), jnp.float32),
            jax.ShapeDtypeStruct((8, D), jnp.float32),
        ),
        scratch_shapes=[pltpu.VMEM((8, D), jnp.float32)],
    )(ain, aout, rin, rout, cin, cout, h, win, wout, wloop, sm)


def _tc_relstep(r2, w):
    return pl.pallas_call(
        _relstep_kernel,
        out_shape=jax.ShapeDtypeStruct((2 * ROWS_A, D), jnp.float32),
    )(r2, w)


def _tc_bn(pre, stats, sm):
    return pl.pallas_call(
        _bn_kernel,
        out_shape=jax.ShapeDtypeStruct((N, D), jnp.float32),
    )(pre, stats, sm)


def _pack8(*rows):
    out = [jnp.reshape(r, (1, D)) for r in rows]
    out += [jnp.zeros((1, D), jnp.float32)] * (8 - len(out))
    return jnp.concatenate(out, axis=0)


def kernel(x, edge_index, edge_type, quals, feat_W, feat_b, init_rel,
           conv0_w_in, conv0_w_out, conv0_w_loop, conv0_w_rel, conv0_loop_rel,
           conv0_bias, conv0_gamma, conv0_beta,
           conv1_w_in, conv1_w_out, conv1_w_loop, conv1_w_rel, conv1_loop_rel,
           conv1_bias, conv1_gamma, conv1_beta):
    del quals
    ei0 = edge_index[0].astype(jnp.int32)
    ei1 = edge_index[1].astype(jnp.int32)
    et = edge_type.astype(jnp.int32)

    pad = EPW * NS - E
    z_pad = jnp.zeros((pad,), jnp.int32)
    t_pad = jnp.full((pad,), TRASH, jnp.int32)
    c_pad = jnp.full((pad,), GREL_PAD, jnp.int32)
    srcs = jnp.concatenate([ei0, z_pad, ei1, z_pad]).reshape(2 * NS * NCH, 128)
    dsts = jnp.concatenate([ei1, t_pad, ei0, t_pad]).reshape(2 * NS * NCH, 128)
    cols = jnp.concatenate([et, c_pad, et + 256, c_pad]).reshape(2 * NS * NCH, 128)

    zeros_a = jnp.zeros((ROWS_A, D), jnp.float32)
    ones_tab = jnp.ones((512, D), jnp.float32)

    ra0p = jnp.concatenate(
        [init_rel, conv0_loop_rel, jnp.zeros((6, D), jnp.float32)], axis=0)
    zrow56 = jnp.zeros((56, D), jnp.float32)
    grel = jnp.concatenate([ra0p[:200], zrow56, ra0p[200:400], zrow56], axis=0)
    featb8 = _pack8(feat_b)
    l18 = _pack8(conv1_loop_rel[0])

    h0, ra1p, routp = _tc_feat(x, feat_W, featb8, ra0p, conv0_w_rel,
                               conv1_w_rel, l18)

    r0 = _sc_rel(grel, cols, dsts, zeros_a)
    cnt = _sc_cnt(dsts, ones_tab, zeros_a)
    r1 = _tc_relstep(r0, conv0_w_rel)

    rins = (r0[:N], r1[:N])
    routs = (r0[ROWS_A:ROWS_A + N], r1[ROWS_A:ROWS_A + N])
    cin = cnt[:N]
    cout = cnt[ROWS_A:ROWS_A + N]

    h = h0
    params = [
        (conv0_w_in, conv0_w_out, conv0_w_loop, conv0_loop_rel, conv0_bias,
         conv0_gamma, conv0_beta),
        (conv1_w_in, conv1_w_out, conv1_w_loop, conv1_loop_rel,
         conv1_bias, conv1_gamma, conv1_beta),
    ]
    for li, (w_in, w_out, w_loop, loop_rel, bias, gamma, beta) in enumerate(params):
        a = _sc_accum(h, srcs, dsts, zeros_a)
        ain = a[:N]
        aout = a[ROWS_A:ROWS_A + N]
        sm = _pack8(loop_rel[0], bias, gamma, beta)
        pre, stats = _tc_combine(ain, aout, rins[li], routs[li], cin, cout, h,
                                 w_in, w_out, w_loop, sm)
        h = _tc_bn(pre, stats, sm)

    return (h, routp[:401])


# speedup vs baseline: 3.6234x; 1.0004x over previous
import jax
import jax.numpy as jnp
from jax import lax
from jax.experimental import pallas as pl
from jax.experimental.pallas import tpu as pltpu
from jax.experimental.pallas import tpu_sc as plsc

N = 10000
E = 320000
FD = 256
D = 128
NR = 200

NC = 2
NS = 16

EPW = 20480
NCH = EPW // 128
ROWS_A = 10112
SLAB = ROWS_A // NS
TRASH = 10016
GREL_PAD = 500


GRP = 40


def _stream_pass(table, idxs, dsts, rbase, acc,
                 sidx, didx, gbuf0, gbuf1, sem0, sem1):
    for grp in range(NCH // GRP):
        pltpu.sync_copy(idxs.at[pl.ds(rbase + grp * GRP, GRP)], sidx)
        pltpu.sync_copy(dsts.at[pl.ds(rbase + grp * GRP, GRP)], didx)
        pltpu.async_copy(table.at[sidx.at[0]], gbuf0, sem0)
        pltpu.async_copy(table.at[sidx.at[1]], gbuf1, sem1)

        def body(i, carry):
            k0 = 2 * i
            k1 = 2 * i + 1
            pltpu.make_async_copy(table.at[sidx.at[k0]], gbuf0, sem0).wait()
            pltpu.sync_copy(gbuf0, acc.at[didx.at[k0]], add=True)

            @pl.when(k0 + 2 < GRP)
            def _():
                pltpu.async_copy(table.at[sidx.at[k0 + 2]], gbuf0, sem0)

            pltpu.make_async_copy(table.at[sidx.at[k1]], gbuf1, sem1).wait()
            pltpu.sync_copy(gbuf1, acc.at[didx.at[k1]], add=True)

            @pl.when(k1 + 2 < GRP)
            def _():
                pltpu.async_copy(table.at[sidx.at[k1 + 2]], gbuf1, sem1)

            return carry

        lax.fori_loop(0, GRP // 2, body, 0)


def _sc_pre_kernel(h, srcs, dsts, cols, grel, ones_tab, zeros_a,
                   outr, outc, outa,
                   acc, tab, sidx, didx, gbuf0, gbuf1, sem0, sem1):
    c = lax.axis_index("c")
    s = lax.axis_index("s")
    rbase = (c * NS + s) * NCH
    ob = c * ROWS_A + s * SLAB

    def zero_acc():
        pltpu.sync_copy(zeros_a.at[pl.ds(s * SLAB, SLAB)],
                        acc.at[pl.ds(s * SLAB, SLAB)])

    def dump(out):
        pltpu.sync_copy(acc.at[pl.ds(s * SLAB, SLAB)], out.at[pl.ds(ob, SLAB)])

    zero_acc()

    @pl.when(s == 0)
    def _():
        pltpu.sync_copy(grel, tab)

    plsc.subcore_barrier()
    _stream_pass(tab, cols, dsts, rbase, acc,
                 sidx, didx, gbuf0, gbuf1, sem0, sem1)
    plsc.subcore_barrier()
    dump(outr)
    plsc.subcore_barrier()

    zero_acc()
    pltpu.sync_copy(ones_tab.at[pl.ds(0, 128)], gbuf0)
    plsc.subcore_barrier()

    def fire(j, carry):
        pltpu.async_copy(gbuf0, acc.at[didx.at[j]], sem0, add=True)
        return carry

    def drain(j, carry):
        pltpu.make_async_copy(gbuf0, acc.at[didx.at[0]], sem0).wait()
        return carry

    for grp in range(NCH // GRP):
        pltpu.sync_copy(dsts.at[pl.ds(rbase + grp * GRP, GRP)], didx)
        lax.fori_loop(0, GRP, fire, 0)
        lax.fori_loop(0, GRP, drain, 0)
    plsc.subcore_barrier()
    dump(outc)
    plsc.subcore_barrier()

    zero_acc()
    plsc.subcore_barrier()
    _stream_pass(h, srcs, dsts, rbase, acc,
                 sidx, didx, gbuf0, gbuf1, sem0, sem1)
    plsc.subcore_barrier()
    dump(outa)


def _sc_pre(h, srcs, dsts, cols, grel, ones_tab, zeros_a):
    return pl.kernel(
        _sc_pre_kernel,
        out_type=(
            jax.ShapeDtypeStruct((2 * ROWS_A, D), jnp.float32),
            jax.ShapeDtypeStruct((2 * ROWS_A, D), jnp.float32),
            jax.ShapeDtypeStruct((2 * ROWS_A, D), jnp.float32),
        ),
        mesh=plsc.VectorSubcoreMesh(core_axis_name="c", subcore_axis_name="s"),
        scratch_types=[
            pltpu.VMEM_SHARED((ROWS_A, D), jnp.float32),
            pltpu.VMEM_SHARED((512, D), jnp.float32),
            pltpu.VMEM((GRP, 128), jnp.int32),
            pltpu.VMEM((GRP, 128), jnp.int32),
            pltpu.VMEM((128, D), jnp.float32),
            pltpu.VMEM((128, D), jnp.float32),
            pltpu.SemaphoreType.DMA,
            pltpu.SemaphoreType.DMA,
        ],
    )(h, srcs, dsts, cols, grel, ones_tab, zeros_a)


def _sc_accum_kernel(h, srcs, dsts, zeros_a, out, acc,
                     sidx, didx, gbuf0, gbuf1, sem0, sem1):
    c = lax.axis_index("c")
    s = lax.axis_index("s")

    pltpu.sync_copy(zeros_a.at[pl.ds(s * SLAB, SLAB)],
                    acc.at[pl.ds(s * SLAB, SLAB)])
    rbase = (c * NS + s) * NCH
    plsc.subcore_barrier()
    _stream_pass(h, srcs, dsts, rbase, acc,
                 sidx, didx, gbuf0, gbuf1, sem0, sem1)
    plsc.subcore_barrier()

    ob = c * ROWS_A + s * SLAB
    pltpu.sync_copy(acc.at[pl.ds(s * SLAB, SLAB)], out.at[pl.ds(ob, SLAB)])


def _sc_accum(h, srcs, dsts, zeros_a):
    return pl.kernel(
        _sc_accum_kernel,
        out_type=jax.ShapeDtypeStruct((2 * ROWS_A, D), jnp.float32),
        mesh=plsc.VectorSubcoreMesh(core_axis_name="c", subcore_axis_name="s"),
        scratch_types=[
            pltpu.VMEM_SHARED((ROWS_A, D), jnp.float32),
            pltpu.VMEM((GRP, 128), jnp.int32),
            pltpu.VMEM((GRP, 128), jnp.int32),
            pltpu.VMEM((128, D), jnp.float32),
            pltpu.VMEM((128, D), jnp.float32),
            pltpu.SemaphoreType.DMA,
            pltpu.SemaphoreType.DMA,
        ],
    )(h, srcs, dsts, zeros_a)


def _feat_kernel(x_ref, w_ref, b_ref, ra0_ref, wr0_ref, wr1_ref, l1_ref,
                 h_ref, ra1_ref, rout_ref):
    h_ref[...] = jnp.dot(x_ref[...], w_ref[...],
                         preferred_element_type=jnp.float32) + b_ref[0:1, :]
    ra1 = jnp.dot(ra0_ref[...], wr0_ref[...], preferred_element_type=jnp.float32)
    rows = lax.broadcasted_iota(jnp.int32, ra1.shape, 0)
    ra1 = jnp.where(rows == 401, l1_ref[0:1, :], ra1)
    ra1_ref[...] = ra1
    rout_ref[...] = jnp.dot(ra1, wr1_ref[...], preferred_element_type=jnp.float32)


def _combine_kernel(ain_ref, aout_ref, rin_ref, rout_ref, cin_ref, cout_ref,
                    h_ref, win_ref, wout_ref, wloop_ref,
                    sm_ref, pre_ref, stats_ref, acc):
    i = pl.program_id(0)

    @pl.when(i == 0)
    def _():
        acc[...] = jnp.zeros_like(acc)

    ni = jnp.maximum(cin_ref[:, 0:1], 1.0)
    no = jnp.maximum(cout_ref[:, 0:1], 1.0)
    in_res = jnp.dot((ain_ref[...] - rin_ref[...]) / ni, win_ref[...],
                     preferred_element_type=jnp.float32)
    out_res = jnp.dot((aout_ref[...] - rout_ref[...]) / no, wout_ref[...],
                      preferred_element_type=jnp.float32)
    loop_res = jnp.dot(h_ref[...] - sm_ref[0:1, :], wloop_ref[...],
                       preferred_element_type=jnp.float32)
    pre = (in_res + out_res + loop_res) / 3.0 + sm_ref[1:2, :]
    pre_ref[...] = pre
    acc[0:1, :] += jnp.sum(pre, axis=0, keepdims=True)
    acc[1:2, :] += jnp.sum(pre * pre, axis=0, keepdims=True)

    @pl.when(i == pl.num_programs(0) - 1)
    def _():
        stats_ref[...] = acc[...]


def _relstep_kernel(r_ref, w_ref, o_ref):
    o_ref[...] = jnp.dot(r_ref[...], w_ref[...],
                         preferred_element_type=jnp.float32)


def _bn_kernel(pre_ref, stats_ref, sm_ref, h_ref):
    mu = stats_ref[0:1, :] / float(N)
    ex2 = stats_ref[1:2, :] / float(N)
    var = ex2 - mu * mu
    inv = lax.rsqrt(var + 1e-5)
    h = (pre_ref[...] - mu) * inv * sm_ref[2:3, :] + sm_ref[3:4, :]
    h_ref[...] = jnp.maximum(h, 0.0)


def _tc_feat(x, feat_W, featb8, ra0p, wr0, wr1, l18):
    return pl.pallas_call(
        _feat_kernel,
        out_shape=(
            jax.ShapeDtypeStruct((N, D), jnp.float32),
            jax.ShapeDtypeStruct((408, D), jnp.float32),
            jax.ShapeDtypeStruct((408, D), jnp.float32),
        ),
    )(x, feat_W, featb8, ra0p, wr0, wr1, l18)


def _tc_combine(ain, aout, rin, rout, cin, cout, h, win, wout, wloop, sm):
    nb = 10
    blk = N // nb
    row = lambda i: (i, 0)
    full = lambda i: (0, 0)
    return pl.pallas_call(
        _combine_kernel,
        grid=(nb,),
        in_specs=[
            pl.BlockSpec((blk, D), row),
            pl.BlockSpec((blk, D), row),
            pl.BlockSpec((blk, D), row),
            pl.BlockSpec((blk, D), row),
            pl.BlockSpec((blk, D), row),
            pl.BlockSpec((blk, D), row),
            pl.BlockSpec((blk, D), row),
            pl.BlockSpec((D, D), full),
            pl.BlockSpec((D, D), full),
            pl.BlockSpec((D, D), full),
            pl.BlockSpec((8, D), full),
        ],
        out_specs=[
            pl.BlockSpec((blk, D), row),
            pl.BlockSpec((8, D), full),
        ],
        out_shape=(
            jax.ShapeDtypeStruct((N, D), jnp.float32),
            jax.ShapeDtypeStruct((8, D), jnp.float32),
        ),
        scratch_shapes=[pltpu.VMEM((8, D), jnp.float32)],
    )(ain, aout, rin, rout, cin, cout, h, win, wout, wloop, sm)


def _tc_relstep(r2, w):
    return pl.pallas_call(
        _relstep_kernel,
        out_shape=jax.ShapeDtypeStruct((2 * ROWS_A, D), jnp.float32),
    )(r2, w)


def _tc_bn(pre, stats, sm):
    return pl.pallas_call(
        _bn_kernel,
        out_shape=jax.ShapeDtypeStruct((N, D), jnp.float32),
    )(pre, stats, sm)


def _pack8(*rows):
    out = [jnp.reshape(r, (1, D)) for r in rows]
    out += [jnp.zeros((1, D), jnp.float32)] * (8 - len(out))
    return jnp.concatenate(out, axis=0)


def kernel(x, edge_index, edge_type, quals, feat_W, feat_b, init_rel,
           conv0_w_in, conv0_w_out, conv0_w_loop, conv0_w_rel, conv0_loop_rel,
           conv0_bias, conv0_gamma, conv0_beta,
           conv1_w_in, conv1_w_out, conv1_w_loop, conv1_w_rel, conv1_loop_rel,
           conv1_bias, conv1_gamma, conv1_beta):
    del quals
    ei0 = edge_index[0].astype(jnp.int32)
    ei1 = edge_index[1].astype(jnp.int32)
    et = edge_type.astype(jnp.int32)

    pad = EPW * NS - E
    z_pad = jnp.zeros((pad,), jnp.int32)
    t_pad = jnp.full((pad,), TRASH, jnp.int32)
    c_pad = jnp.full((pad,), GREL_PAD, jnp.int32)
    srcs = jnp.concatenate([ei0, z_pad, ei1, z_pad]).reshape(2 * NS * NCH, 128)
    dsts = jnp.concatenate([ei1, t_pad, ei0, t_pad]).reshape(2 * NS * NCH, 128)
    cols = jnp.concatenate([et, c_pad, et + 256, c_pad]).reshape(2 * NS * NCH, 128)

    zeros_a = jnp.zeros((ROWS_A, D), jnp.float32)
    ones_tab = jnp.ones((512, D), jnp.float32)

    ra0p = jnp.concatenate(
        [init_rel, conv0_loop_rel, jnp.zeros((6, D), jnp.float32)], axis=0)
    zrow56 = jnp.zeros((56, D), jnp.float32)
    grel = jnp.concatenate([ra0p[:200], zrow56, ra0p[200:400], zrow56], axis=0)
    featb8 = _pack8(feat_b)
    l18 = _pack8(conv1_loop_rel[0])

    h0, ra1p, routp = _tc_feat(x, feat_W, featb8, ra0p, conv0_w_rel,
                               conv1_w_rel, l18)

    r0, cnt, a0 = _sc_pre(h0, srcs, dsts, cols, grel, ones_tab, zeros_a)
    r1 = _tc_relstep(r0, conv0_w_rel)

    rins = (r0[:N], r1[:N])
    routs = (r0[ROWS_A:ROWS_A + N], r1[ROWS_A:ROWS_A + N])
    cin = cnt[:N]
    cout = cnt[ROWS_A:ROWS_A + N]

    h = h0
    params = [
        (conv0_w_in, conv0_w_out, conv0_w_loop, conv0_loop_rel, conv0_bias,
         conv0_gamma, conv0_beta),
        (conv1_w_in, conv1_w_out, conv1_w_loop, conv1_loop_rel,
         conv1_bias, conv1_gamma, conv1_beta),
    ]
    for li, (w_in, w_out, w_loop, loop_rel, bias, gamma, beta) in enumerate(params):
        a = a0 if li == 0 else _sc_accum(h, srcs, dsts, zeros_a)
        ain = a[:N]
        aout = a[ROWS_A:ROWS_A + N]
        sm = _pack8(loop_rel[0], bias, gamma, beta)
        pre, stats = _tc_combine(ain, aout, rins[li], routs[li], cin, cout, h,
                                 w_in, w_out, w_loop, sm)
        h = _tc_bn(pre, stats, sm)

    return (h, routp[:401])
